# baseline (speedup 1.0000x reference)
import jax
import jax.numpy as jnp
from jax import lax
from jax.experimental import pallas as pl
from jax.experimental.pallas import tpu as pltpu
from jax.experimental.pallas import tpu_sc as plsc

BATCH = 1024
VOCAB = 100000
K = int(0.7 * BATCH)

TC_ROWS = 512
SC_ROWS = BATCH - TC_ROWS
RB = 16
NI_TC = TC_ROWS // RB

NSUB = 32
RPS = SC_ROWS // NSUB
CH = 20000
NCH = VOCAB // CH
NFLAT = RPS * NCH
VPC = CH // 16
UNROLL = 10


def _tc_loss_body(x_ref, t_ref, loss_ref):
    x = x_ref[...]
    col = lax.broadcasted_iota(jnp.int32, (RB, VOCAB), 1)
    m = jnp.max(x, axis=1, keepdims=True)
    s = jnp.sum(jnp.exp(x - m), axis=1, keepdims=True)
    tv = jnp.sum(jnp.where(col == t_ref[...], x, 0.0), axis=1, keepdims=True)
    loss_ref[...] = m + jnp.log(s) - tv


def _tc_loss(x, tgt2d):
    return pl.pallas_call(
        _tc_loss_body,
        grid=(NI_TC,),
        in_specs=[
            pl.BlockSpec((RB, VOCAB), lambda i: (i, 0)),
            pl.BlockSpec((RB, 1), lambda i: (i, 0)),
        ],
        out_specs=pl.BlockSpec((RB, 1), lambda i: (i, 0)),
        out_shape=jax.ShapeDtypeStruct((TC_ROWS, 1), jnp.float32),
        compiler_params=pltpu.CompilerParams(
            dimension_semantics=("arbitrary",)),
    )(x, tgt2d)


def _chunk_pass(buf, m_vec, s_vec):
    def max_body(j, mc):
        for u in range(UNROLL):
            mc = jnp.maximum(mc, buf[pl.ds((j * UNROLL + u) * 16, 16)])
        return mc
    mc = lax.fori_loop(0, VPC // UNROLL, max_body,
                       jnp.full((16,), -jnp.inf, jnp.float32))
    m_new = jnp.maximum(m_vec, mc)

    def sum_body(j, sc):
        for u in range(UNROLL):
            sc = sc + jnp.exp(buf[pl.ds((j * UNROLL + u) * 16, 16)] - m_new)
        return sc
    sc = lax.fori_loop(0, VPC // UNROLL, sum_body,
                       jnp.zeros((16,), jnp.float32))
    s_new = s_vec * jnp.exp(m_vec - m_new) + sc
    return m_new, s_new


def _sc_stream_body(x_hbm, tgt_hbm, out_hbm, buf0, buf1, tgt_v, out_v, sems):
    c = lax.axis_index("c")
    s = lax.axis_index("s")
    wid = s * 2 + c
    row0 = TC_ROWS + wid * RPS
    lane = lax.broadcasted_iota(jnp.int32, (16,), 0)

    pltpu.sync_copy(tgt_hbm.at[pl.ds(TC_ROWS + wid * RPS, RPS)], tgt_v)
    tv = tgt_v[...]

    def copy(n, buf, sem_i):
        row = row0 + n // NCH
        k = lax.rem(n, NCH)
        return pltpu.make_async_copy(
            x_hbm.at[row, pl.ds(k * CH, CH)], buf, sems.at[sem_i])

    def start(n, buf, sem_i):
        @pl.when(n < NFLAT)
        def _():
            copy(n, buf, sem_i).start()

    start(jnp.int32(0), buf0, 0)
    start(jnp.int32(1), buf1, 1)

    def process(n, buf, sem_i, carry):
        m_vec, s_vec, tl_vec, m_acc, s_acc = carry
        copy(n, buf, sem_i).wait()
        k = lax.rem(n, NCH)
        r_loc = n // NCH
        m_new, s_new = _chunk_pass(buf, m_vec, s_vec)
        g = plsc.load_gather(
            buf, [jnp.clip(tv - k * CH, 0, CH - 1)])
        hit = jnp.logical_and(lane == r_loc,
                              jnp.logical_and(tv >= k * CH, tv < k * CH + CH))
        tl_vec = jnp.where(hit, g, tl_vec)
        is_last = k == NCH - 1
        m_r = jnp.max(m_new)
        s_r = jnp.sum(s_new * jnp.exp(m_new - m_r))
        sel = jnp.logical_and(is_last, lane == r_loc)
        m_acc = jnp.where(sel, m_r, m_acc)
        s_acc = jnp.where(sel, s_r, s_acc)
        m_vec = jnp.where(is_last, jnp.full((16,), -jnp.inf, jnp.float32),
                          m_new)
        s_vec = jnp.where(is_last, jnp.zeros((16,), jnp.float32), s_new)
        return m_vec, s_vec, tl_vec, m_acc, s_acc

    def pair_body(p, carry):
        n = 2 * p
        carry = process(n, buf0, 0, carry)
        start(n + 2, buf0, 0)
        carry = process(n + 1, buf1, 1, carry)
        start(n + 3, buf1, 1)
        return carry

    init = (jnp.full((16,), -jnp.inf, jnp.float32),
            jnp.zeros((16,), jnp.float32),
            jnp.zeros((16,), jnp.float32),
            jnp.zeros((16,), jnp.float32),
            jnp.ones((16,), jnp.float32))
    _, _, tl_vec, m_acc, s_acc = lax.fori_loop(0, NFLAT // 2, pair_body, init)

    b = plsc.bitcast(s_acc, jnp.int32)
    y = (b - 0x3F800000).astype(jnp.float32) * jnp.float32(8.262958e-8)
    for _ in range(3):
        y = y + s_acc * jnp.exp(-y) - 1.0
    out_v[...] = m_acc + y - tl_vec
    pltpu.sync_copy(out_v, out_hbm.at[pl.ds(wid * RPS, RPS)])


def _sc_loss(x, tgt1d):
    fn = pl.kernel(
        _sc_stream_body,
        out_type=jax.ShapeDtypeStruct((SC_ROWS,), jnp.float32),
        mesh=plsc.VectorSubcoreMesh(core_axis_name="c", subcore_axis_name="s"),
        scratch_types=[
            pltpu.VMEM((CH,), jnp.float32),
            pltpu.VMEM((CH,), jnp.float32),
            pltpu.VMEM((RPS,), jnp.int32),
            pltpu.VMEM((16,), jnp.float32),
            pltpu.SemaphoreType.DMA((2,)),
        ],
        compiler_params=pltpu.CompilerParams(
            needs_layout_passes=False, use_tc_tiling_on_sc=False),
    )
    return fn(x, tgt1d)


_NVR = BATCH // 16
_I32_MIN = -2147483648
_I32_FLIP = 0x7FFFFFFF


def _topk_body(loss_hbm, out_hbm, loss_v, key_v, out_v):
    c = lax.axis_index("c")
    s = lax.axis_index("s")

    @pl.when(jnp.logical_and(c == 0, s == 0))
    def _work():
        pltpu.sync_copy(loss_hbm, loss_v)

        for i in range(_NVR):
            b = plsc.bitcast(loss_v[pl.ds(i * 16, 16)], jnp.int32)
            key_v[pl.ds(i * 16, 16)] = jnp.where(b < 0, b ^ _I32_FLIP, b)

        def count_ge(cand):
            acc = jnp.zeros((16,), jnp.int32)
            for i in range(_NVR):
                kv = key_v[pl.ds(i * 16, 16)]
                acc = acc + jnp.where(kv >= cand, 1, 0).astype(jnp.int32)
            return jnp.sum(acc)

        t0 = jnp.where(count_ge(jnp.int32(0)) >= K,
                       jnp.int32(0), jnp.int32(_I32_MIN))

        def bit_step(i, t):
            cand = t | lax.shift_left(jnp.int32(1), jnp.int32(30) - i)
            return jnp.where(count_ge(cand) >= K, cand, t)

        t = lax.fori_loop(0, 31, bit_step, t0)

        acc_sum = jnp.zeros((16,), jnp.float32)
        acc_cnt = jnp.zeros((16,), jnp.int32)
        for i in range(_NVR):
            kv = key_v[pl.ds(i * 16, 16)]
            xv = loss_v[pl.ds(i * 16, 16)]
            m = kv > t
            acc_sum = acc_sum + jnp.where(m, xv, 0.0)
            acc_cnt = acc_cnt + jnp.where(m, 1, 0).astype(jnp.int32)
        sum_gt = jnp.sum(acc_sum)
        cnt_gt = jnp.sum(acc_cnt)

        tbits = jnp.where(t < 0, t ^ _I32_FLIP, t)
        tval = plsc.bitcast(jnp.full((16,), tbits, jnp.int32), jnp.float32)
        mean_vec = (sum_gt + (K - cnt_gt).astype(jnp.float32) * tval) * (1.0 / K)
        out_v[...] = mean_vec
        pltpu.sync_copy(out_v, out_hbm)


def _topk_mean(loss1d):
    fn = pl.kernel(
        _topk_body,
        out_type=jax.ShapeDtypeStruct((16,), jnp.float32),
        mesh=plsc.VectorSubcoreMesh(core_axis_name="c", subcore_axis_name="s"),
        scratch_types=[
            pltpu.VMEM((BATCH,), jnp.float32),
            pltpu.VMEM((BATCH,), jnp.int32),
            pltpu.VMEM((16,), jnp.float32),
        ],
        compiler_params=pltpu.CompilerParams(needs_layout_passes=False),
    )
    return fn(loss1d)


def kernel(x, target):
    tgt = target.astype(jnp.int32)
    loss_tc = _tc_loss(x, tgt[:TC_ROWS].reshape(TC_ROWS, 1))
    loss_sc = _sc_loss(x, tgt)
    loss = jnp.concatenate([loss_tc.reshape(TC_ROWS), loss_sc], axis=0)
    return _topk_mean(loss)[0]


# speedup vs baseline: 1.0848x; 1.0848x over previous
import jax
import jax.numpy as jnp
from jax import lax
from jax.experimental import pallas as pl
from jax.experimental.pallas import tpu as pltpu
from jax.experimental.pallas import tpu_sc as plsc

BATCH = 1024
VOCAB = 100000
K = int(0.7 * BATCH)

C0 = 65536
RB = 16
NI_TC = BATCH // RB
WTC = 2048
NJ_TC = -(-(VOCAB - C0) // WTC)
JB0 = C0 // WTC

NSUB = 32
RPSUB = BATCH // NSUB
NBAND = RPSUB // 8
CW = 4096
NCH = C0 // CW
NFLAT = NBAND * NCH
VPR = CW // 16
UNR = 8


def _tc_body(x_ref, t_ref, m_ref, s_ref, tv_ref, m_acc, s_acc, t_acc):
    j = pl.program_id(1)

    @pl.when(j == 0)
    def _init():
        m_acc[...] = jnp.full((RB, 1), -jnp.inf, jnp.float32)
        s_acc[...] = jnp.zeros((RB, 1), jnp.float32)
        t_acc[...] = jnp.zeros((RB, 1), jnp.float32)

    x = x_ref[...]
    col = lax.broadcasted_iota(jnp.int32, (RB, WTC), 1)
    limit = VOCAB - (C0 + j * WTC)
    xm = jnp.where(col < limit, x, -jnp.inf)
    mj = jnp.max(xm, axis=1, keepdims=True)
    m_old = m_acc[...]
    m_new = jnp.maximum(m_old, mj)
    sj = jnp.sum(jnp.exp(xm - m_new), axis=1, keepdims=True)
    s_acc[...] = s_acc[...] * jnp.exp(m_old - m_new) + sj
    m_acc[...] = m_new
    tshift = t_ref[...] - (C0 + j * WTC)
    t_acc[...] = t_acc[...] + jnp.sum(
        jnp.where(col == tshift, xm, 0.0), axis=1, keepdims=True)

    @pl.when(j == NJ_TC - 1)
    def _fin():
        m_ref[...] = m_acc[...]
        s_ref[...] = s_acc[...]
        tv_ref[...] = t_acc[...]


def _tc_partial(x, tgt2d):
    return pl.pallas_call(
        _tc_body,
        grid=(NI_TC, NJ_TC),
        in_specs=[
            pl.BlockSpec((RB, WTC), lambda i, j: (i, JB0 + j)),
            pl.BlockSpec((RB, 1), lambda i, j: (i, 0)),
        ],
        out_specs=[
            pl.BlockSpec((RB, 1), lambda i, j: (i, 0)),
            pl.BlockSpec((RB, 1), lambda i, j: (i, 0)),
            pl.BlockSpec((RB, 1), lambda i, j: (i, 0)),
        ],
        out_shape=[jax.ShapeDtypeStruct((BATCH, 1), jnp.float32)] * 3,
        scratch_shapes=[pltpu.VMEM((RB, 1), jnp.float32)] * 3,
        compiler_params=pltpu.CompilerParams(
            dimension_semantics=("arbitrary", "arbitrary")),
    )(x, tgt2d)


def _sc_body(x_hbm, tgt_hbm, m_hbm, s_hbm, t_hbm,
             buf0, buf1, tgt_v, mo_v, so_v, to_v, sems):
    c = lax.axis_index("c")
    s = lax.axis_index("s")
    wid = s * 2 + c
    row0 = wid * RPSUB
    lane = lax.broadcasted_iota(jnp.int32, (16,), 0)

    pltpu.sync_copy(tgt_hbm, tgt_v)

    def copy(n, buf, sem_i):
        band = n // NCH
        k = lax.rem(n, NCH)
        return pltpu.make_async_copy(
            x_hbm.at[pl.ds(row0 + band * 8, 8), pl.ds(k * CW, CW)],
            buf, sems.at[sem_i])

    def start(n, buf, sem_i):
        @pl.when(n < NFLAT)
        def _():
            copy(n, buf, sem_i).start()

    start(jnp.int32(0), buf0, 0)
    start(jnp.int32(1), buf1, 1)

    def process(n, buf, sem_i, carry):
        ms, ss, m_lo, m_hi, s_lo, s_hi, t_lo, t_hi = carry
        copy(n, buf, sem_i).wait()
        band = n // NCH
        k = lax.rem(n, NCH)
        base = k * CW
        first = k == 0
        last = k == NCH - 1
        tb16 = tgt_v[pl.ds(row0 + (band // 2) * 16, 16)]

        new_ms, new_ss = [], []
        for r in range(8):
            m_r = jnp.where(first, jnp.full((16,), -jnp.inf, jnp.float32),
                            ms[r])
            s_r = jnp.where(first, jnp.zeros((16,), jnp.float32), ss[r])

            def max_body(j, mc, r=r):
                for u in range(UNR):
                    mc = jnp.maximum(
                        mc, buf[r, pl.ds((j * UNR + u) * 16, 16)])
                return mc
            mc = lax.fori_loop(0, VPR // UNR, max_body,
                               jnp.full((16,), -jnp.inf, jnp.float32))
            m_new = jnp.maximum(m_r, mc)

            def sum_body(j, sc, r=r, m_new=m_new):
                for u in range(UNR):
                    sc = sc + jnp.exp(
                        buf[r, pl.ds((j * UNR + u) * 16, 16)] - m_new)
                return sc
            sc = lax.fori_loop(0, VPR // UNR, sum_body,
                               jnp.zeros((16,), jnp.float32))
            s_new = s_r * jnp.exp(m_r - m_new) + sc

            lane_idx = lax.rem(band, 2) * 8 + r
            t_r = jnp.sum(jnp.where(lane == lane_idx, tb16, 0))

            def t_scan(r=r, t_r=t_r, base=base):
                def body(j, acc):
                    colv = base + j * 16 + lane
                    v = buf[r, pl.ds(j * 16, 16)]
                    return acc + jnp.where(colv == t_r, v, 0.0)
                accv = lax.fori_loop(0, VPR, body,
                                     jnp.zeros((16,), jnp.float32))
                return jnp.sum(accv)

            hit = jnp.logical_and(t_r >= base, t_r < base + CW)
            t_val = lax.cond(hit, t_scan, lambda: jnp.float32(0.0))

            g_idx = band * 8 + r
            sel_lo = jnp.logical_and(last,
                                     jnp.logical_and(g_idx < 16,
                                                     lane == g_idx))
            sel_hi = jnp.logical_and(last,
                                     jnp.logical_and(g_idx >= 16,
                                                     lane == g_idx - 16))
            m_row = jnp.max(m_new)
            s_row = jnp.sum(s_new * jnp.exp(m_new - m_row))
            m_lo = jnp.where(sel_lo, m_row, m_lo)
            m_hi = jnp.where(sel_hi, m_row, m_hi)
            s_lo = jnp.where(sel_lo, s_row, s_lo)
            s_hi = jnp.where(sel_hi, s_row, s_hi)
            in_lo = g_idx < 16
            add_lo = jnp.where(
                jnp.logical_and(in_lo, lane == g_idx), t_val, 0.0)
            add_hi = jnp.where(
                jnp.logical_and(jnp.logical_not(in_lo), lane == g_idx - 16),
                t_val, 0.0)
            t_lo = t_lo + add_lo
            t_hi = t_hi + add_hi
            new_ms.append(m_new)
            new_ss.append(s_new)

        return (tuple(new_ms), tuple(new_ss),
                m_lo, m_hi, s_lo, s_hi, t_lo, t_hi)

    def pair_body(p, carry):
        n = 2 * p
        carry = process(n, buf0, 0, carry)
        start(n + 2, buf0, 0)
        carry = process(n + 1, buf1, 1, carry)
        start(n + 3, buf1, 1)
        return carry

    zeros = jnp.zeros((16,), jnp.float32)
    ninf = jnp.full((16,), -jnp.inf, jnp.float32)
    init = (tuple(ninf for _ in range(8)), tuple(zeros for _ in range(8)),
            zeros, zeros, zeros, zeros, zeros, zeros)
    res = lax.fori_loop(0, NFLAT // 2, pair_body, init)
    _, _, m_lo, m_hi, s_lo, s_hi, t_lo, t_hi = res

    mo_v[pl.ds(0, 16)] = m_lo
    mo_v[pl.ds(16, 16)] = m_hi
    so_v[pl.ds(0, 16)] = s_lo
    so_v[pl.ds(16, 16)] = s_hi
    to_v[pl.ds(0, 16)] = t_lo
    to_v[pl.ds(16, 16)] = t_hi
    pltpu.sync_copy(mo_v, m_hbm.at[pl.ds(row0, RPSUB)])
    pltpu.sync_copy(so_v, s_hbm.at[pl.ds(row0, RPSUB)])
    pltpu.sync_copy(to_v, t_hbm.at[pl.ds(row0, RPSUB)])


def _sc_partial(x, tgt1d):
    fn = pl.kernel(
        _sc_body,
        out_type=[jax.ShapeDtypeStruct((BATCH,), jnp.float32)] * 3,
        mesh=plsc.VectorSubcoreMesh(core_axis_name="c", subcore_axis_name="s"),
        scratch_types=[
            pltpu.VMEM((8, CW), jnp.float32),
            pltpu.VMEM((8, CW), jnp.float32),
            pltpu.VMEM((BATCH,), jnp.int32),
            pltpu.VMEM((RPSUB,), jnp.float32),
            pltpu.VMEM((RPSUB,), jnp.float32),
            pltpu.VMEM((RPSUB,), jnp.float32),
            pltpu.SemaphoreType.DMA((2,)),
        ],
        compiler_params=pltpu.CompilerParams(needs_layout_passes=False),
    )
    return fn(x, tgt1d)


_NVR = BATCH // 16
_I32_MIN = -2147483648
_I32_FLIP = 0x7FFFFFFF


def _log_newton(sv):
    b = plsc.bitcast(sv, jnp.int32)
    y = (b - 0x3F800000).astype(jnp.float32) * jnp.float32(8.262958e-8)
    for _ in range(3):
        y = y + sv * jnp.exp(-y) - 1.0
    return y


def _topk_body(m1_hbm, s1_hbm, t1_hbm, m2_hbm, s2_hbm, t2_hbm, out_hbm,
               a_v, b_v, c_v, d_v, e_v, f_v, loss_v, key_v, out_v):
    c = lax.axis_index("c")
    s = lax.axis_index("s")

    @pl.when(jnp.logical_and(c == 0, s == 0))
    def _work():
        pltpu.sync_copy(m1_hbm, a_v)
        pltpu.sync_copy(s1_hbm, b_v)
        pltpu.sync_copy(t1_hbm, c_v)
        pltpu.sync_copy(m2_hbm, d_v)
        pltpu.sync_copy(s2_hbm, e_v)
        pltpu.sync_copy(t2_hbm, f_v)

        for i in range(_NVR):
            sl = pl.ds(i * 16, 16)
            m1, s1, t1 = a_v[sl], b_v[sl], c_v[sl]
            m2, s2, t2 = d_v[sl], e_v[sl], f_v[sl]
            mm = jnp.maximum(m1, m2)
            ss = s1 * jnp.exp(m1 - mm) + s2 * jnp.exp(m2 - mm)
            loss = mm + _log_newton(ss) - (t1 + t2)
            loss_v[sl] = loss
            bb = plsc.bitcast(loss, jnp.int32)
            key_v[sl] = jnp.where(bb < 0, bb ^ _I32_FLIP, bb)

        def count_ge(cand):
            acc = jnp.zeros((16,), jnp.int32)
            for i in range(_NVR):
                kv = key_v[pl.ds(i * 16, 16)]
                acc = acc + jnp.where(kv >= cand, 1, 0).astype(jnp.int32)
            return jnp.sum(acc)

        t0 = jnp.where(count_ge(jnp.int32(0)) >= K,
                       jnp.int32(0), jnp.int32(_I32_MIN))

        def bit_step(i, t):
            cand = t | lax.shift_left(jnp.int32(1), jnp.int32(30) - i)
            return jnp.where(count_ge(cand) >= K, cand, t)

        t = lax.fori_loop(0, 31, bit_step, t0)

        acc_sum = jnp.zeros((16,), jnp.float32)
        acc_cnt = jnp.zeros((16,), jnp.int32)
        for i in range(_NVR):
            kv = key_v[pl.ds(i * 16, 16)]
            xv = loss_v[pl.ds(i * 16, 16)]
            m = kv > t
            acc_sum = acc_sum + jnp.where(m, xv, 0.0)
            acc_cnt = acc_cnt + jnp.where(m, 1, 0).astype(jnp.int32)
        sum_gt = jnp.sum(acc_sum)
        cnt_gt = jnp.sum(acc_cnt)

        tbits = jnp.where(t < 0, t ^ _I32_FLIP, t)
        tval = plsc.bitcast(jnp.full((16,), tbits, jnp.int32), jnp.float32)
        mean_vec = (sum_gt + (K - cnt_gt).astype(jnp.float32) * tval) * (1.0 / K)
        out_v[...] = mean_vec
        pltpu.sync_copy(out_v, out_hbm)


def _topk_mean(m1, s1, t1, m2, s2, t2):
    fn = pl.kernel(
        _topk_body,
        out_type=jax.ShapeDtypeStruct((16,), jnp.float32),
        mesh=plsc.VectorSubcoreMesh(core_axis_name="c", subcore_axis_name="s"),
        scratch_types=[pltpu.VMEM((BATCH,), jnp.float32)] * 6 + [
            pltpu.VMEM((BATCH,), jnp.float32),
            pltpu.VMEM((BATCH,), jnp.int32),
            pltpu.VMEM((16,), jnp.float32),
        ],
        compiler_params=pltpu.CompilerParams(needs_layout_passes=False),
    )
    return fn(m1, s1, t1, m2, s2, t2)


def kernel(x, target):
    tgt = target.astype(jnp.int32)
    m2, s2, t2 = _tc_partial(x, tgt.reshape(BATCH, 1))
    m1, s1, t1 = _sc_partial(x, tgt)
    out16 = _topk_mean(m1, s1, t1,
                       m2.reshape(BATCH), s2.reshape(BATCH),
                       t2.reshape(BATCH))
    return out16[0]


# speedup vs baseline: 6.1413x; 5.6610x over previous
import jax
import jax.numpy as jnp
from jax import lax
from jax.experimental import pallas as pl
from jax.experimental.pallas import tpu as pltpu
from jax.experimental.pallas import tpu_sc as plsc

BATCH = 1024
VOCAB = 100000
K = int(0.7 * BATCH)
WV = 2048
NJ = -(-VOCAB // WV)


def _loss_body(xt_ref, t_ref, loss_ref, m_acc, s_acc, t_acc):
    j = pl.program_id(0)

    @pl.when(j == 0)
    def _init():
        m_acc[...] = jnp.full((1, BATCH), -jnp.inf, jnp.float32)
        s_acc[...] = jnp.zeros((1, BATCH), jnp.float32)
        t_acc[...] = jnp.zeros((1, BATCH), jnp.float32)

    x = xt_ref[...]
    row = lax.broadcasted_iota(jnp.int32, (WV, BATCH), 0)
    limit = VOCAB - j * WV
    xm = jnp.where(row < limit, x, -jnp.inf)

    mj = jnp.max(xm, axis=0, keepdims=True)
    m_old = m_acc[...]
    m_new = jnp.maximum(m_old, mj)
    sj = jnp.sum(jnp.exp(xm - m_new), axis=0, keepdims=True)
    s_acc[...] = s_acc[...] * jnp.exp(m_old - m_new) + sj
    m_acc[...] = m_new

    tshift = t_ref[...] - j * WV
    t_acc[...] = t_acc[...] + jnp.sum(
        jnp.where(row == tshift, xm, 0.0), axis=0, keepdims=True)

    @pl.when(j == NJ - 1)
    def _fin():
        loss_ref[...] = m_acc[...] + jnp.log(s_acc[...]) - t_acc[...]


def _per_sample_loss(xt, tgt2d):
    return pl.pallas_call(
        _loss_body,
        grid=(NJ,),
        in_specs=[
            pl.BlockSpec((WV, BATCH), lambda j: (j, 0)),
            pl.BlockSpec((1, BATCH), lambda j: (0, 0)),
        ],
        out_specs=pl.BlockSpec((1, BATCH), lambda j: (0, 0)),
        out_shape=jax.ShapeDtypeStruct((1, BATCH), jnp.float32),
        scratch_shapes=[pltpu.VMEM((1, BATCH), jnp.float32)] * 3,
        compiler_params=pltpu.CompilerParams(
            dimension_semantics=("arbitrary",)),
    )(xt, tgt2d)


_NVR = BATCH // 16
_I32_MIN = -2147483648
_I32_FLIP = 0x7FFFFFFF


def _topk_body(loss_hbm, out_hbm, loss_v, key_v, out_v):
    c = lax.axis_index("c")
    s = lax.axis_index("s")

    @pl.when(jnp.logical_and(c == 0, s == 0))
    def _work():
        pltpu.sync_copy(loss_hbm, loss_v)

        for i in range(_NVR):
            b = plsc.bitcast(loss_v[pl.ds(i * 16, 16)], jnp.int32)
            key_v[pl.ds(i * 16, 16)] = jnp.where(b < 0, b ^ _I32_FLIP, b)

        def count_ge(cand):
            acc = jnp.zeros((16,), jnp.int32)
            for i in range(_NVR):
                kv = key_v[pl.ds(i * 16, 16)]
                acc = acc + jnp.where(kv >= cand, 1, 0).astype(jnp.int32)
            return jnp.sum(acc)

        t0 = jnp.where(count_ge(jnp.int32(0)) >= K,
                       jnp.int32(0), jnp.int32(_I32_MIN))

        def bit_step(i, t):
            cand = t | lax.shift_left(jnp.int32(1), jnp.int32(30) - i)
            return jnp.where(count_ge(cand) >= K, cand, t)

        t = lax.fori_loop(0, 31, bit_step, t0)

        acc_sum = jnp.zeros((16,), jnp.float32)
        acc_cnt = jnp.zeros((16,), jnp.int32)
        for i in range(_NVR):
            kv = key_v[pl.ds(i * 16, 16)]
            xv = loss_v[pl.ds(i * 16, 16)]
            m = kv > t
            acc_sum = acc_sum + jnp.where(m, xv, 0.0)
            acc_cnt = acc_cnt + jnp.where(m, 1, 0).astype(jnp.int32)
        sum_gt = jnp.sum(acc_sum)
        cnt_gt = jnp.sum(acc_cnt)

        tbits = jnp.where(t < 0, t ^ _I32_FLIP, t)
        tval = plsc.bitcast(jnp.full((16,), tbits, jnp.int32), jnp.float32)
        mean_vec = (sum_gt + (K - cnt_gt).astype(jnp.float32) * tval) * (1.0 / K)
        out_v[...] = mean_vec
        pltpu.sync_copy(out_v, out_hbm)


def _topk_mean(loss1d):
    fn = pl.kernel(
        _topk_body,
        out_type=jax.ShapeDtypeStruct((16,), jnp.float32),
        mesh=plsc.VectorSubcoreMesh(core_axis_name="c", subcore_axis_name="s"),
        scratch_types=[
            pltpu.VMEM((BATCH,), jnp.float32),
            pltpu.VMEM((BATCH,), jnp.int32),
            pltpu.VMEM((16,), jnp.float32),
        ],
        compiler_params=pltpu.CompilerParams(needs_layout_passes=False),
    )
    return fn(loss1d)


def kernel(x, target):
    xt = x.T
    tgt2d = target.astype(jnp.int32).reshape(1, BATCH)
    loss = _per_sample_loss(xt, tgt2d)
    return _topk_mean(loss.reshape(BATCH))[0]


# speedup vs baseline: 6.4201x; 1.0454x over previous
import jax
import jax.numpy as jnp
from jax import lax
from jax.experimental import pallas as pl
from jax.experimental.pallas import tpu as pltpu
from jax.experimental.pallas import tpu_sc as plsc

BATCH = 1024
VOCAB = 100000
K = int(0.7 * BATCH)
WV = 4096
NJ = -(-VOCAB // WV)


def _loss_body(xt_ref, t_ref, loss_ref, m_acc, s_acc, t_acc):
    j = pl.program_id(0)

    @pl.when(j == 0)
    def _init():
        m_acc[...] = jnp.full((1, BATCH), -jnp.inf, jnp.float32)
        s_acc[...] = jnp.zeros((1, BATCH), jnp.float32)
        t_acc[...] = jnp.zeros((1, BATCH), jnp.float32)

    x = xt_ref[...]
    row = lax.broadcasted_iota(jnp.int32, (WV, BATCH), 0)
    limit = VOCAB - j * WV
    xm = jnp.where(row < limit, x, -jnp.inf)

    mj = jnp.max(xm, axis=0, keepdims=True)
    m_old = m_acc[...]
    m_new = jnp.maximum(m_old, mj)
    sj = jnp.sum(jnp.exp(xm - m_new), axis=0, keepdims=True)
    s_acc[...] = s_acc[...] * jnp.exp(m_old - m_new) + sj
    m_acc[...] = m_new

    tshift = t_ref[...] - j * WV
    t_acc[...] = t_acc[...] + jnp.sum(
        jnp.where(row == tshift, xm, 0.0), axis=0, keepdims=True)

    @pl.when(j == NJ - 1)
    def _fin():
        loss_ref[...] = m_acc[...] + jnp.log(s_acc[...]) - t_acc[...]


def _per_sample_loss(xt, tgt2d):
    return pl.pallas_call(
        _loss_body,
        grid=(NJ,),
        in_specs=[
            pl.BlockSpec((WV, BATCH), lambda j: (j, 0)),
            pl.BlockSpec((1, BATCH), lambda j: (0, 0)),
        ],
        out_specs=pl.BlockSpec((1, BATCH), lambda j: (0, 0)),
        out_shape=jax.ShapeDtypeStruct((1, BATCH), jnp.float32),
        scratch_shapes=[pltpu.VMEM((1, BATCH), jnp.float32)] * 3,
        compiler_params=pltpu.CompilerParams(
            dimension_semantics=("arbitrary",),
            vmem_limit_bytes=100 * 1024 * 1024),
    )(xt, tgt2d)


_NVR = BATCH // 16
_I32_MIN = -2147483648
_I32_FLIP = 0x7FFFFFFF


def _topk_body(loss_hbm, out_hbm, loss_v, key_v, out_v):
    c = lax.axis_index("c")
    s = lax.axis_index("s")

    @pl.when(jnp.logical_and(c == 0, s == 0))
    def _work():
        pltpu.sync_copy(loss_hbm, loss_v)

        for i in range(_NVR):
            b = plsc.bitcast(loss_v[pl.ds(i * 16, 16)], jnp.int32)
            key_v[pl.ds(i * 16, 16)] = jnp.where(b < 0, b ^ _I32_FLIP, b)

        def count_ge(cand):
            acc = jnp.zeros((16,), jnp.int32)
            for i in range(_NVR):
                kv = key_v[pl.ds(i * 16, 16)]
                acc = acc + jnp.where(kv >= cand, 1, 0).astype(jnp.int32)
            return jnp.sum(acc)

        t0 = jnp.where(count_ge(jnp.int32(0)) >= K,
                       jnp.int32(0), jnp.int32(_I32_MIN))

        def bit_step(i, t):
            cand = t | lax.shift_left(jnp.int32(1), jnp.int32(30) - i)
            return jnp.where(count_ge(cand) >= K, cand, t)

        t = lax.fori_loop(0, 31, bit_step, t0)

        acc_sum = jnp.zeros((16,), jnp.float32)
        acc_cnt = jnp.zeros((16,), jnp.int32)
        for i in range(_NVR):
            kv = key_v[pl.ds(i * 16, 16)]
            xv = loss_v[pl.ds(i * 16, 16)]
            m = kv > t
            acc_sum = acc_sum + jnp.where(m, xv, 0.0)
            acc_cnt = acc_cnt + jnp.where(m, 1, 0).astype(jnp.int32)
        sum_gt = jnp.sum(acc_sum)
        cnt_gt = jnp.sum(acc_cnt)

        tbits = jnp.where(t < 0, t ^ _I32_FLIP, t)
        tval = plsc.bitcast(jnp.full((16,), tbits, jnp.int32), jnp.float32)
        mean_vec = (sum_gt + (K - cnt_gt).astype(jnp.float32) * tval) * (1.0 / K)
        out_v[...] = mean_vec
        pltpu.sync_copy(out_v, out_hbm)


def _topk_mean(loss1d):
    fn = pl.kernel(
        _topk_body,
        out_type=jax.ShapeDtypeStruct((16,), jnp.float32),
        mesh=plsc.VectorSubcoreMesh(core_axis_name="c", subcore_axis_name="s"),
        scratch_types=[
            pltpu.VMEM((BATCH,), jnp.float32),
            pltpu.VMEM((BATCH,), jnp.int32),
            pltpu.VMEM((16,), jnp.float32),
        ],
        compiler_params=pltpu.CompilerParams(needs_layout_passes=False),
    )
    return fn(loss1d)


def kernel(x, target):
    xt = x.T
    tgt2d = target.astype(jnp.int32).reshape(1, BATCH)
    loss = _per_sample_loss(xt, tgt2d)
    return _topk_mean(loss.reshape(BATCH))[0]
